# baseline (speedup 1.0000x reference)
import functools

import jax
import jax.numpy as jnp
import numpy as np
from jax.experimental import pallas as pl

TOPK = 8
WIN = 25
T = 512
NFFT = 1024
K = NFFT // 2 + 1
BLOCK = 64


def _constants():
    pad = WIN // 2
    M = np.zeros((T, T), dtype=np.float64)
    for tau in range(T):
        for j in range(-pad, pad + 1):
            src = min(max(tau + j, 0), T - 1)
            M[src, tau] += 1.0 / WIN
    A = np.eye(T, dtype=np.float64) - M

    t = np.arange(T, dtype=np.int64)[:, None]
    k = np.arange(K, dtype=np.int64)[None, :]
    ang = 2.0 * np.pi * ((t * k) % NFFT).astype(np.float64) / NFFT
    C = np.cos(ang)
    S = np.sin(ang)

    kk = np.arange(K, dtype=np.int64)[:, None]
    tt = np.arange(T, dtype=np.int64)[None, :]
    ang2 = 2.0 * np.pi * ((kk * tt) % NFFT).astype(np.float64) / NFFT
    w = np.full((K, 1), 2.0 / NFFT, dtype=np.float64)
    w[0, 0] = 1.0 / NFFT
    w[K - 1, 0] = 1.0 / NFFT
    Ci = np.cos(ang2) * w

    f32 = lambda a: jnp.asarray(a, dtype=jnp.float32)
    return f32(A), f32(C), f32(S), f32(Ci)


def _body(x_ref, a_ref, c_ref, s_ref, ci_ref, out_ref):
    dot = functools.partial(
        jax.lax.dot,
        precision=jax.lax.Precision.HIGHEST,
        preferred_element_type=jnp.float32,
    )
    x = jnp.mean(x_ref[...], axis=-1)
    d = dot(x, a_ref[...])
    re = dot(d, c_ref[...])
    im = dot(d, s_ref[...])
    p = re * re + im * im
    ac = dot(p, ci_ref[...])

    lane = jax.lax.broadcasted_iota(jnp.int32, ac.shape, 1)
    ac = jnp.where(lane == 0, jnp.float32(-1e9), ac)

    b = ac.shape[0]
    out_lane = jax.lax.broadcasted_iota(jnp.int32, (b, TOPK), 1)
    out = jnp.zeros((b, TOPK), dtype=jnp.int32)
    work = ac
    for kth in range(TOPK):
        m = jnp.max(work, axis=1, keepdims=True)
        arg = jnp.min(
            jnp.where(work == m, lane, jnp.int32(T)), axis=1, keepdims=True
        )
        out = jnp.where(out_lane == kth, jnp.broadcast_to(arg, (b, TOPK)), out)
        work = jnp.where(lane == arg, jnp.float32(-3e38), work)
    out_ref[...] = out


def kernel(X):
    BN, t, d = X.shape
    A, C, S, Ci = _constants()
    grid = (BN // BLOCK,)
    return pl.pallas_call(
        _body,
        grid=grid,
        in_specs=[
            pl.BlockSpec((BLOCK, t, d), lambda i: (i, 0, 0)),
            pl.BlockSpec((T, T), lambda i: (0, 0)),
            pl.BlockSpec((T, K), lambda i: (0, 0)),
            pl.BlockSpec((T, K), lambda i: (0, 0)),
            pl.BlockSpec((K, T), lambda i: (0, 0)),
        ],
        out_specs=pl.BlockSpec((BLOCK, TOPK), lambda i: (i, 0)),
        out_shape=jax.ShapeDtypeStruct((BN, TOPK), jnp.int32),
    )(X, A, C, S, Ci)


# speedup vs baseline: 1.0024x; 1.0024x over previous
import functools

import jax
import jax.numpy as jnp
import numpy as np
from jax.experimental import pallas as pl

TOPK = 8
WIN = 25
T = 512
NFFT = 1024
K = NFFT // 2 + 1
BLOCK = 32


def _constants():
    pad = WIN // 2
    M = np.zeros((T, T), dtype=np.float64)
    for tau in range(T):
        for j in range(-pad, pad + 1):
            src = min(max(tau + j, 0), T - 1)
            M[src, tau] += 1.0 / WIN
    A = np.eye(T, dtype=np.float64) - M

    t = np.arange(T, dtype=np.int64)[:, None]
    k = np.arange(K, dtype=np.int64)[None, :]
    ang = 2.0 * np.pi * ((t * k) % NFFT).astype(np.float64) / NFFT
    C = np.cos(ang)
    S = np.sin(ang)

    kk = np.arange(K, dtype=np.int64)[:, None]
    tt = np.arange(T, dtype=np.int64)[None, :]
    ang2 = 2.0 * np.pi * ((kk * tt) % NFFT).astype(np.float64) / NFFT
    w = np.full((K, 1), 2.0 / NFFT, dtype=np.float64)
    w[0, 0] = 1.0 / NFFT
    w[K - 1, 0] = 1.0 / NFFT
    Ci = np.cos(ang2) * w

    f32 = lambda a: jnp.asarray(a, dtype=jnp.float32)
    return f32(A), f32(C), f32(S), f32(Ci)


def _body(x1_ref, x2_ref, a_ref, c_ref, s_ref, ci_ref, out1_ref, out2_ref):
    dot = functools.partial(
        jax.lax.dot,
        precision=jax.lax.Precision.HIGHEST,
        preferred_element_type=jnp.float32,
    )
    x1 = jnp.mean(x1_ref[...], axis=-1)
    x2 = jnp.mean(x2_ref[...], axis=-1)
    x = jnp.concatenate([x1, x2], axis=0)
    d = dot(x, a_ref[...])
    re = dot(d, c_ref[...])
    im = dot(d, s_ref[...])
    p = re * re + im * im
    ac = dot(p, ci_ref[...])

    lane = jax.lax.broadcasted_iota(jnp.int32, ac.shape, 1)
    ac = jnp.where(lane == 0, jnp.float32(-1e9), ac)

    b = ac.shape[0]
    out_lane = jax.lax.broadcasted_iota(jnp.int32, (b, TOPK), 1)
    out = jnp.zeros((b, TOPK), dtype=jnp.int32)
    work = ac
    for kth in range(TOPK):
        m = jnp.max(work, axis=1, keepdims=True)
        arg = jnp.min(
            jnp.where(work == m, lane, jnp.int32(T)), axis=1, keepdims=True
        )
        out = jnp.where(out_lane == kth, jnp.broadcast_to(arg, (b, TOPK)), out)
        work = jnp.where(lane == arg, jnp.float32(-3e38), work)
    out1_ref[...] = out[:BLOCK]
    out2_ref[...] = out[BLOCK:]


def kernel(X):
    BN, t, d = X.shape
    A, C, S, Ci = _constants()
    half = BN // 2
    grid = (half // BLOCK,)
    out1, out2 = pl.pallas_call(
        _body,
        grid=grid,
        in_specs=[
            pl.BlockSpec((BLOCK, t, d), lambda i: (i, 0, 0)),
            pl.BlockSpec((BLOCK, t, d), lambda i: (i + half // BLOCK, 0, 0)),
            pl.BlockSpec((T, T), lambda i: (0, 0)),
            pl.BlockSpec((T, K), lambda i: (0, 0)),
            pl.BlockSpec((T, K), lambda i: (0, 0)),
            pl.BlockSpec((K, T), lambda i: (0, 0)),
        ],
        out_specs=[
            pl.BlockSpec((BLOCK, TOPK), lambda i: (i, 0)),
            pl.BlockSpec((BLOCK, TOPK), lambda i: (i, 0)),
        ],
        out_shape=[
            jax.ShapeDtypeStruct((half, TOPK), jnp.int32),
            jax.ShapeDtypeStruct((half, TOPK), jnp.int32),
        ],
    )(X, X, A, C, S, Ci)
    return jnp.concatenate([out1, out2], axis=0)
